# Initial kernel scaffold; baseline (speedup 1.0000x reference)
import functools

import jax
import jax.numpy as jnp
from jax import lax
from jax.experimental import pallas as pl
from jax.experimental.pallas import tpu as pltpu
from jax.experimental.pallas import tpu_sc as plsc

N = 10000
E = 320000
D = 128
ALPHA = 0.01
KSTEPS = 10

NUM_TILES = 16
EDGES_PER_TILE = E // NUM_TILES
ROWS_PER_TILE = N // NUM_TILES
ECHUNK = 80
NECHUNKS = EDGES_PER_TILE // ECHUNK
RCHUNK = 125
NRCHUNKS = ROWS_PER_TILE // RCHUNK
LANES = 16
JROW = D // LANES

SCALE_IN = ALPHA / (1.0 - ALPHA)
SCALE_OUT = 1.0 - ALPHA


def _mlp_body(x_ref, w1_ref, b1_ref, w2_ref, b2_ref, o_ref):
    h = jnp.dot(x_ref[...], w1_ref[...],
                preferred_element_type=jnp.float32,
                precision=lax.Precision.HIGHEST)
    h = jnp.maximum(h + b1_ref[...], 0.0)
    o = jnp.dot(h, w2_ref[...],
                preferred_element_type=jnp.float32,
                precision=lax.Precision.HIGHEST)
    o_ref[...] = o + b2_ref[...]


def _mlp(x, W1, b1, W2, b2):
    return pl.pallas_call(
        _mlp_body,
        out_shape=jax.ShapeDtypeStruct((N, D), jnp.float32),
        grid=(10,),
        in_specs=[
            pl.BlockSpec((N // 10, D), lambda i: (i, 0)),
            pl.BlockSpec((D, D), lambda i: (0, 0)),
            pl.BlockSpec((1, D), lambda i: (0, 0)),
            pl.BlockSpec((D, D), lambda i: (0, 0)),
            pl.BlockSpec((1, D), lambda i: (0, 0)),
        ],
        out_specs=pl.BlockSpec((N // 10, D), lambda i: (i, 0)),
    )(x, W1, b1.reshape(1, D), W2, b2.reshape(1, D))


def _prop_body(h_hbm, row_hbm, col_hbm, val_hbm, za_hbm, zb_hbm,
               agg, cbuf, rbuf, vbuf, gbuf, sbuf, sem):
    sid = lax.axis_index("s")
    ebase = sid * EDGES_PER_TILE
    rbase = sid * ROWS_PER_TILE

    for k in range(KSTEPS):
        src = h_hbm if k == 0 else (za_hbm if k % 2 == 1 else zb_hbm)
        dst = za_hbm if k % 2 == 0 else zb_hbm

        def init_chunk(i, _):
            r0 = rbase + i * RCHUNK
            pltpu.sync_copy(src.at[pl.ds(r0, RCHUNK)], sbuf)

            def scale_row(r, _):
                for j in range(JROW):
                    sl = (r, pl.ds(j * LANES, LANES))
                    sbuf[sl] = sbuf[sl] * SCALE_IN
                return 0

            lax.fori_loop(0, RCHUNK, scale_row, 0)
            pltpu.sync_copy(sbuf, agg.at[pl.ds(r0, RCHUNK)])
            return 0

        lax.fori_loop(0, NRCHUNKS, init_chunk, 0)
        plsc.subcore_barrier()

        def edge_chunk(g, _):
            e0 = ebase + g * ECHUNK
            pltpu.sync_copy(col_hbm.at[pl.ds(e0, ECHUNK)], cbuf)
            pltpu.sync_copy(row_hbm.at[pl.ds(e0, ECHUNK)], rbuf)
            pltpu.sync_copy(val_hbm.at[pl.ds(e0, ECHUNK)], vbuf)
            pltpu.async_copy(src.at[cbuf], gbuf, sem).wait()

            def scale_edge(e, _):
                v = vbuf[e]
                for j in range(JROW):
                    sl = (e, pl.ds(j * LANES, LANES))
                    gbuf[sl] = gbuf[sl] * v
                return 0

            lax.fori_loop(0, ECHUNK, scale_edge, 0)
            pltpu.sync_copy(gbuf, agg.at[rbuf], add=True)
            return 0

        lax.fori_loop(0, NECHUNKS, edge_chunk, 0)
        plsc.subcore_barrier()

        def comb_chunk(i, _):
            r0 = rbase + i * RCHUNK
            pltpu.sync_copy(agg.at[pl.ds(r0, RCHUNK)], sbuf)

            def scale_row(r, _):
                for j in range(JROW):
                    sl = (r, pl.ds(j * LANES, LANES))
                    sbuf[sl] = sbuf[sl] * SCALE_OUT
                return 0

            lax.fori_loop(0, RCHUNK, scale_row, 0)
            pltpu.sync_copy(sbuf, dst.at[pl.ds(r0, RCHUNK)])
            return 0

        lax.fori_loop(0, NRCHUNKS, comb_chunk, 0)
        plsc.subcore_barrier()


_prop = pl.kernel(
    _prop_body,
    out_type=(
        jax.ShapeDtypeStruct((N, D), jnp.float32),
        jax.ShapeDtypeStruct((N, D), jnp.float32),
    ),
    mesh=plsc.VectorSubcoreMesh(
        core_axis_name="c", subcore_axis_name="s", num_cores=1),
    scratch_types=[
        pltpu.VMEM_SHARED((N, D), jnp.float32),
        pltpu.VMEM((ECHUNK,), jnp.int32),
        pltpu.VMEM((ECHUNK,), jnp.int32),
        pltpu.VMEM((ECHUNK,), jnp.float32),
        pltpu.VMEM((ECHUNK, D), jnp.float32),
        pltpu.VMEM((RCHUNK, D), jnp.float32),
        pltpu.SemaphoreType.DMA,
    ],
)


@jax.jit
def kernel(x, edge_index, edge_vals, W1, b1, W2, b2):
    h = _mlp(x, W1, b1, W2, b2)
    row = edge_index[0]
    col = edge_index[1]
    za, zb = _prop(h, row, col, edge_vals)
    return zb


# speedup vs baseline: 1.9949x; 1.9949x over previous
import functools

import jax
import jax.numpy as jnp
from jax import lax
from jax.experimental import pallas as pl
from jax.experimental.pallas import tpu as pltpu
from jax.experimental.pallas import tpu_sc as plsc

N = 10000
E = 320000
D = 128
ALPHA = 0.01
KSTEPS = 10

NUM_TILES = 16
EDGES_PER_TILE = E // NUM_TILES
ROWS_PER_TILE = N // NUM_TILES
ECHUNK = 80
NECHUNKS = EDGES_PER_TILE // ECHUNK
RCHUNK = 200
NRCHUNKS = N // RCHUNK
LANES = 16
JROW = D // LANES

SCALE_IN = ALPHA / (1.0 - ALPHA)
SCALE_OUT = 1.0 - ALPHA


def _mlp_body(x_ref, w1_ref, b1_ref, w2_ref, b2_ref, o_ref):
    h = jnp.dot(x_ref[...], w1_ref[...],
                preferred_element_type=jnp.float32,
                precision=lax.Precision.HIGHEST)
    h = jnp.maximum(h + b1_ref[...], 0.0)
    o = jnp.dot(h, w2_ref[...],
                preferred_element_type=jnp.float32,
                precision=lax.Precision.HIGHEST)
    o_ref[...] = o + b2_ref[...]


def _mlp(x, W1, b1, W2, b2):
    return pl.pallas_call(
        _mlp_body,
        out_shape=jax.ShapeDtypeStruct((N, D), jnp.float32),
        grid=(10,),
        in_specs=[
            pl.BlockSpec((N // 10, D), lambda i: (i, 0)),
            pl.BlockSpec((D, D), lambda i: (0, 0)),
            pl.BlockSpec((1, D), lambda i: (0, 0)),
            pl.BlockSpec((D, D), lambda i: (0, 0)),
            pl.BlockSpec((1, D), lambda i: (0, 0)),
        ],
        out_specs=pl.BlockSpec((N // 10, D), lambda i: (i, 0)),
    )(x, W1, b1.reshape(1, D), W2, b2.reshape(1, D))


def _prop_body(h_hbm, row_hbm, col_hbm, val_hbm, za_hbm, zb_hbm,
               agg, cbuf, rbuf, vbuf, gbuf, sbuf, sem):
    sid = lax.axis_index("s")
    ebase = sid * EDGES_PER_TILE
    nrc_mine = (NRCHUNKS - sid + NUM_TILES - 1) // NUM_TILES

    for k in range(KSTEPS):
        src = h_hbm if k == 0 else (za_hbm if k % 2 == 1 else zb_hbm)
        dst = za_hbm if k % 2 == 0 else zb_hbm

        def init_chunk(i, _):
            r0 = (i * NUM_TILES + sid) * RCHUNK
            pltpu.sync_copy(src.at[pl.ds(r0, RCHUNK)], sbuf)

            def scale_row(r, _):
                for j in range(JROW):
                    sl = (r, pl.ds(j * LANES, LANES))
                    sbuf[sl] = sbuf[sl] * SCALE_IN
                return 0

            lax.fori_loop(0, RCHUNK, scale_row, 0)
            pltpu.sync_copy(sbuf, agg.at[pl.ds(r0, RCHUNK)])
            return 0

        lax.fori_loop(0, nrc_mine, init_chunk, 0)
        plsc.subcore_barrier()

        def edge_chunk(g, _):
            e0 = ebase + g * ECHUNK
            pltpu.sync_copy(col_hbm.at[pl.ds(e0, ECHUNK)], cbuf)
            pltpu.sync_copy(row_hbm.at[pl.ds(e0, ECHUNK)], rbuf)
            pltpu.sync_copy(val_hbm.at[pl.ds(e0, ECHUNK)], vbuf)
            pltpu.async_copy(src.at[cbuf], gbuf, sem).wait()

            def scale_group(gq, _):
                vbase = gq * LANES
                v16 = vbuf[pl.ds(vbase, LANES)]
                for i in range(LANES):
                    v = v16[i]
                    for j in range(JROW):
                        sl = (vbase + i, pl.ds(j * LANES, LANES))
                        gbuf[sl] = gbuf[sl] * v
                return 0

            lax.fori_loop(0, ECHUNK // LANES, scale_group, 0)
            pltpu.sync_copy(gbuf, agg.at[rbuf], add=True)
            return 0

        lax.fori_loop(0, NECHUNKS, edge_chunk, 0)
        plsc.subcore_barrier()

        def comb_chunk(i, _):
            r0 = (i * NUM_TILES + sid) * RCHUNK
            pltpu.sync_copy(agg.at[pl.ds(r0, RCHUNK)], sbuf)

            def scale_row(r, _):
                for j in range(JROW):
                    sl = (r, pl.ds(j * LANES, LANES))
                    sbuf[sl] = sbuf[sl] * SCALE_OUT
                return 0

            lax.fori_loop(0, RCHUNK, scale_row, 0)
            pltpu.sync_copy(sbuf, dst.at[pl.ds(r0, RCHUNK)])
            return 0

        lax.fori_loop(0, nrc_mine, comb_chunk, 0)
        plsc.subcore_barrier()


_prop = pl.kernel(
    _prop_body,
    out_type=(
        jax.ShapeDtypeStruct((N, D), jnp.float32),
        jax.ShapeDtypeStruct((N, D), jnp.float32),
    ),
    mesh=plsc.VectorSubcoreMesh(
        core_axis_name="c", subcore_axis_name="s", num_cores=1),
    scratch_types=[
        pltpu.VMEM_SHARED((N, D), jnp.float32),
        pltpu.VMEM((ECHUNK,), jnp.int32),
        pltpu.VMEM((ECHUNK,), jnp.int32),
        pltpu.VMEM((ECHUNK,), jnp.float32),
        pltpu.VMEM((ECHUNK, D), jnp.float32),
        pltpu.VMEM((RCHUNK, D), jnp.float32),
        pltpu.SemaphoreType.DMA,
    ],
)


@jax.jit
def kernel(x, edge_index, edge_vals, W1, b1, W2, b2):
    h = _mlp(x, W1, b1, W2, b2)
    row = edge_index[0]
    col = edge_index[1]
    za, zb = _prop(h, row, col, edge_vals)
    return zb


# speedup vs baseline: 2.6629x; 1.3349x over previous
import functools

import jax
import jax.numpy as jnp
from jax import lax
from jax.experimental import pallas as pl
from jax.experimental.pallas import tpu as pltpu
from jax.experimental.pallas import tpu_sc as plsc

N = 10000
E = 320000
D = 128
ALPHA = 0.01
KSTEPS = 10

LANES = 16
JROW = D // LANES
ECHUNK = 128
PACK = 3 * ECHUNK

NT = 16
NCH = -(-(-(-E // (ECHUNK * NT))) // 8) * 8
NCHT = NCH * NT
E_PAD = NCHT * ECHUNK
RCH = 80
NRCH = N // RCH

SCALE_IN = ALPHA / (1.0 - ALPHA)
SCALE_OUT = 1.0 - ALPHA


def _mlp_body(x_ref, w1_ref, b1_ref, w2_ref, b2_ref, o_ref):
    h = jnp.dot(x_ref[...], w1_ref[...],
                preferred_element_type=jnp.float32,
                precision=lax.Precision.HIGHEST)
    h = jnp.maximum(h + b1_ref[...], 0.0)
    o = jnp.dot(h, w2_ref[...],
                preferred_element_type=jnp.float32,
                precision=lax.Precision.HIGHEST)
    o_ref[...] = o + b2_ref[...]


def _mlp(x, W1, b1, W2, b2):
    return pl.pallas_call(
        _mlp_body,
        out_shape=jax.ShapeDtypeStruct((N, D), jnp.float32),
        grid=(10,),
        in_specs=[
            pl.BlockSpec((N // 10, D), lambda i: (i, 0)),
            pl.BlockSpec((D, D), lambda i: (0, 0)),
            pl.BlockSpec((1, D), lambda i: (0, 0)),
            pl.BlockSpec((D, D), lambda i: (0, 0)),
            pl.BlockSpec((1, D), lambda i: (0, 0)),
        ],
        out_specs=pl.BlockSpec((N // 10, D), lambda i: (i, 0)),
    )(x, W1, b1.reshape(1, D), W2, b2.reshape(1, D))


def _prop_body(h_hbm, packed_hbm, zz_hbm,
               agg, gb0, gb1, pv0, pv1, cbuf0, cbuf1, rb0, rb1, sbuf,
               se0, se1, sg0, sg1, ss0, ss1):
    sid = lax.axis_index("s")
    cbase = sid * NCH
    nrc_mine = (NRCH - sid + NT - 1) // NT

    gbs = (gb0, gb1)
    pvs = (pv0, pv1)
    cbufs = (cbuf0, cbuf1)
    rbs = (rb0, rb1)
    ses = (se0, se1)
    sgs = (sg0, sg1)
    sss = (ss0, ss1)

    def hcopy(i, _):
        r0 = (i * NT + sid) * RCH
        pltpu.sync_copy(h_hbm.at[pl.ds(r0, RCH)], sbuf)
        pltpu.sync_copy(sbuf, zz_hbm.at[pl.ds(r0, RCH)])
        return 0
    lax.fori_loop(0, nrc_mine, hcopy, 0)
    plsc.subcore_barrier()

    def kstep(k, _):
        koff = (k % 2) * N
        doff = N - koff

        def seed_chunk(i, _):
            r0 = (i * NT + sid) * RCH
            pltpu.sync_copy(zz_hbm.at[pl.ds(koff + r0, RCH)], sbuf)

            def srow(r, _):
                for j in range(JROW):
                    sl = (r, pl.ds(j * LANES, LANES))
                    sbuf[sl] = sbuf[sl] * SCALE_IN
                return 0
            lax.fori_loop(0, RCH, srow, 0)
            pltpu.sync_copy(sbuf, agg.at[pl.ds(r0, RCH)])
            return 0
        lax.fori_loop(0, nrc_mine, seed_chunk, 0)
        plsc.subcore_barrier()

        def build_cbuf(cb, pv):
            for j in range(JROW):
                sl = pl.ds(j * LANES, LANES)
                cb[sl] = pv[sl] + koff
            return cb

        pltpu.sync_copy(packed_hbm.at[pl.ds(cbase * PACK, PACK)], pv0)
        build_cbuf(cbuf0, pv0)
        pltpu.async_copy(zz_hbm.at[cbuf0], gb0, sg0)
        pltpu.async_copy(packed_hbm.at[pl.ds((cbase + 1) * PACK, PACK)],
                         pv1, se1)

        def pair_body(p, _):
            for sb in range(2):
                c = 2 * p + sb
                gb, pv, cb, rb = gbs[sb], pvs[sb], cbufs[sb], rbs[sb]
                gbn, pvn, cbn, rbn = (gbs[1 - sb], pvs[1 - sb],
                                      cbufs[1 - sb], rbs[1 - sb])
                sg, se, ss = sgs[sb], ses[sb], sss[sb]
                sgn, sen, ssn = sgs[1 - sb], ses[1 - sb], sss[1 - sb]

                @pl.when(c + 1 < NCH)
                def _():
                    pltpu.make_async_copy(
                        packed_hbm.at[pl.ds((cbase + c + 1) * PACK, PACK)],
                        pvn, sen).wait()
                    build_cbuf(cbn, pvn)

                    @pl.when(c >= 1)
                    def _():
                        pltpu.make_async_copy(gbn, agg.at[rbn], ssn).wait()

                    pltpu.async_copy(zz_hbm.at[cbn], gbn, sgn)

                pltpu.make_async_copy(zz_hbm.at[cb], gb, sg).wait()

                def group(q, _):
                    qb = q * LANES
                    rb[pl.ds(qb, LANES)] = pv[pl.ds(ECHUNK + qb, LANES)]
                    v16 = lax.bitcast_convert_type(
                        pv[pl.ds(2 * ECHUNK + qb, LANES)], jnp.float32)
                    for i in range(LANES):
                        v = v16[i]
                        for j in range(JROW):
                            sl = (qb + i, pl.ds(j * LANES, LANES))
                            gb[sl] = gb[sl] * v
                    return 0
                lax.fori_loop(0, ECHUNK // LANES, group, 0)

                pltpu.async_copy(gb, agg.at[rb], ss, add=True)

                @pl.when(c + 2 < NCH)
                def _():
                    pltpu.async_copy(
                        packed_hbm.at[pl.ds((cbase + c + 2) * PACK, PACK)],
                        pv, se)
            return 0

        lax.fori_loop(0, NCH // 2, pair_body, 0)
        pltpu.make_async_copy(gb0, agg.at[rb0], ss0).wait()
        pltpu.make_async_copy(gb1, agg.at[rb1], ss1).wait()
        plsc.subcore_barrier()

        def comb_chunk(i, _):
            r0 = (i * NT + sid) * RCH
            pltpu.sync_copy(agg.at[pl.ds(r0, RCH)], sbuf)

            def srow(r, _):
                for j in range(JROW):
                    sl = (r, pl.ds(j * LANES, LANES))
                    sbuf[sl] = sbuf[sl] * SCALE_OUT
                return 0
            lax.fori_loop(0, RCH, srow, 0)
            pltpu.sync_copy(sbuf, zz_hbm.at[pl.ds(doff + r0, RCH)])
            return 0
        lax.fori_loop(0, nrc_mine, comb_chunk, 0)
        plsc.subcore_barrier()
        return 0

    lax.fori_loop(0, KSTEPS, kstep, 0)


_prop = pl.kernel(
    _prop_body,
    out_type=jax.ShapeDtypeStruct((2 * N, D), jnp.float32),
    mesh=plsc.VectorSubcoreMesh(
        core_axis_name="c", subcore_axis_name="s",
        num_cores=1, num_subcores=16),
    scratch_types=[
        pltpu.VMEM_SHARED((N, D), jnp.float32),
        pltpu.VMEM((ECHUNK, D), jnp.float32),
        pltpu.VMEM((ECHUNK, D), jnp.float32),
        pltpu.VMEM((PACK,), jnp.int32),
        pltpu.VMEM((PACK,), jnp.int32),
        pltpu.VMEM((ECHUNK,), jnp.int32),
        pltpu.VMEM((ECHUNK,), jnp.int32),
        pltpu.VMEM((ECHUNK,), jnp.int32),
        pltpu.VMEM((ECHUNK,), jnp.int32),
        pltpu.VMEM((RCH, D), jnp.float32),
        pltpu.SemaphoreType.DMA,
        pltpu.SemaphoreType.DMA,
        pltpu.SemaphoreType.DMA,
        pltpu.SemaphoreType.DMA,
        pltpu.SemaphoreType.DMA,
        pltpu.SemaphoreType.DMA,
    ],
)


@jax.jit
def kernel(x, edge_index, edge_vals, W1, b1, W2, b2):
    h = _mlp(x, W1, b1, W2, b2)
    row = edge_index[0]
    col = edge_index[1]
    pad = E_PAD - E
    rowp = jnp.concatenate([row, jnp.zeros((pad,), jnp.int32)])
    colp = jnp.concatenate([col, jnp.zeros((pad,), jnp.int32)])
    valp = jnp.concatenate([edge_vals, jnp.zeros((pad,), jnp.float32)])
    packed = jnp.stack(
        [colp.reshape(NCHT, ECHUNK),
         rowp.reshape(NCHT, ECHUNK),
         lax.bitcast_convert_type(valp, jnp.int32).reshape(NCHT, ECHUNK)],
        axis=1).reshape(-1)
    zz = _prop(h, packed)
    return zz[:N]


# speedup vs baseline: 2.7030x; 1.0150x over previous
import functools

import jax
import jax.numpy as jnp
from jax import lax
from jax.experimental import pallas as pl
from jax.experimental.pallas import tpu as pltpu
from jax.experimental.pallas import tpu_sc as plsc

N = 10000
E = 320000
D = 128
ALPHA = 0.01
KSTEPS = 10

LANES = 16
JROW = D // LANES
ECHUNK = 128
PACK = 3 * ECHUNK

NT = 16
NCH = -(-(-(-E // (ECHUNK * NT))) // 8) * 8
NCHT = NCH * NT
E_PAD = NCHT * ECHUNK
RCH = 80
NRCH = N // RCH

SCALE_IN = ALPHA / (1.0 - ALPHA)
SCALE_OUT = 1.0 - ALPHA


def _mlp_body(x_ref, w1_ref, b1_ref, w2_ref, b2_ref, o_ref):
    h = jnp.dot(x_ref[...], w1_ref[...],
                preferred_element_type=jnp.float32,
                precision=lax.Precision.HIGHEST)
    h = jnp.maximum(h + b1_ref[...], 0.0)
    o = jnp.dot(h, w2_ref[...],
                preferred_element_type=jnp.float32,
                precision=lax.Precision.HIGHEST)
    o_ref[...] = o + b2_ref[...]


def _mlp(x, W1, b1, W2, b2):
    return pl.pallas_call(
        _mlp_body,
        out_shape=jax.ShapeDtypeStruct((N, D), jnp.float32),
        grid=(10,),
        in_specs=[
            pl.BlockSpec((N // 10, D), lambda i: (i, 0)),
            pl.BlockSpec((D, D), lambda i: (0, 0)),
            pl.BlockSpec((1, D), lambda i: (0, 0)),
            pl.BlockSpec((D, D), lambda i: (0, 0)),
            pl.BlockSpec((1, D), lambda i: (0, 0)),
        ],
        out_specs=pl.BlockSpec((N // 10, D), lambda i: (i, 0)),
    )(x, W1, b1.reshape(1, D), W2, b2.reshape(1, D))


def _prop_body(h_hbm, packed_hbm, zz_hbm,
               agg, gb0, gb1, pv0, pv1, cbuf0, cbuf1, rb0, rb1, sbuf,
               se0, se1, sg0, sg1, ss0, ss1):
    sid = lax.axis_index("s")
    cbase = sid * NCH
    nrc_mine = (NRCH - sid + NT - 1) // NT

    gbs = (gb0, gb1)
    pvs = (pv0, pv1)
    cbufs = (cbuf0, cbuf1)
    rbs = (rb0, rb1)
    ses = (se0, se1)
    sgs = (sg0, sg1)
    sss = (ss0, ss1)

    def hcopy(i, _):
        r0 = (i * NT + sid) * RCH
        pltpu.sync_copy(h_hbm.at[pl.ds(r0, RCH)], sbuf)
        pltpu.sync_copy(sbuf, zz_hbm.at[pl.ds(r0, RCH)])

        def srow(r, _):
            for j in range(JROW):
                sl = (r, pl.ds(j * LANES, LANES))
                sbuf[sl] = sbuf[sl] * SCALE_IN
            return 0
        lax.fori_loop(0, RCH, srow, 0)
        pltpu.sync_copy(sbuf, agg.at[pl.ds(r0, RCH)])
        return 0
    lax.fori_loop(0, nrc_mine, hcopy, 0)
    plsc.subcore_barrier()

    def kstep(k, _):
        koff = (k % 2) * N
        doff = N - koff

        def build_cbuf(cb, pv):
            for j in range(JROW):
                sl = pl.ds(j * LANES, LANES)
                cb[sl] = pv[sl] + koff
            return cb

        pltpu.sync_copy(packed_hbm.at[pl.ds(cbase * PACK, PACK)], pv0)
        build_cbuf(cbuf0, pv0)
        pltpu.async_copy(zz_hbm.at[cbuf0], gb0, sg0)
        pltpu.async_copy(packed_hbm.at[pl.ds((cbase + 1) * PACK, PACK)],
                         pv1, se1)

        def pair_body(p, _):
            for sb in range(2):
                c = 2 * p + sb
                gb, pv, cb, rb = gbs[sb], pvs[sb], cbufs[sb], rbs[sb]
                gbn, pvn, cbn, rbn = (gbs[1 - sb], pvs[1 - sb],
                                      cbufs[1 - sb], rbs[1 - sb])
                sg, se, ss = sgs[sb], ses[sb], sss[sb]
                sgn, sen, ssn = sgs[1 - sb], ses[1 - sb], sss[1 - sb]

                @pl.when(c + 1 < NCH)
                def _():
                    pltpu.make_async_copy(
                        packed_hbm.at[pl.ds((cbase + c + 1) * PACK, PACK)],
                        pvn, sen).wait()
                    build_cbuf(cbn, pvn)

                    @pl.when(c >= 1)
                    def _():
                        pltpu.make_async_copy(gbn, agg.at[rbn], ssn).wait()

                    pltpu.async_copy(zz_hbm.at[cbn], gbn, sgn)

                pltpu.make_async_copy(zz_hbm.at[cb], gb, sg).wait()

                def group(q, _):
                    qb = q * LANES
                    rb[pl.ds(qb, LANES)] = pv[pl.ds(ECHUNK + qb, LANES)]
                    v16 = lax.bitcast_convert_type(
                        pv[pl.ds(2 * ECHUNK + qb, LANES)], jnp.float32)
                    for i in range(LANES):
                        v = v16[i]
                        for j in range(JROW):
                            sl = (qb + i, pl.ds(j * LANES, LANES))
                            gb[sl] = gb[sl] * v
                    return 0
                lax.fori_loop(0, ECHUNK // LANES, group, 0)

                pltpu.async_copy(gb, agg.at[rb], ss, add=True)

                @pl.when(c + 2 < NCH)
                def _():
                    pltpu.async_copy(
                        packed_hbm.at[pl.ds((cbase + c + 2) * PACK, PACK)],
                        pv, se)
            return 0

        lax.fori_loop(0, NCH // 2, pair_body, 0)
        pltpu.make_async_copy(gb0, agg.at[rb0], ss0).wait()
        pltpu.make_async_copy(gb1, agg.at[rb1], ss1).wait()
        plsc.subcore_barrier()

        def comb_chunk(i, _):
            r0 = (i * NT + sid) * RCH
            pltpu.sync_copy(agg.at[pl.ds(r0, RCH)], sbuf)

            def srow(r, _):
                for j in range(JROW):
                    sl = (r, pl.ds(j * LANES, LANES))
                    sbuf[sl] = sbuf[sl] * SCALE_OUT
                return 0
            lax.fori_loop(0, RCH, srow, 0)
            pltpu.sync_copy(sbuf, zz_hbm.at[pl.ds(doff + r0, RCH)])

            def srow2(r, _):
                for j in range(JROW):
                    sl = (r, pl.ds(j * LANES, LANES))
                    sbuf[sl] = sbuf[sl] * SCALE_IN
                return 0
            lax.fori_loop(0, RCH, srow2, 0)
            pltpu.sync_copy(sbuf, agg.at[pl.ds(r0, RCH)])
            return 0
        lax.fori_loop(0, nrc_mine, comb_chunk, 0)
        plsc.subcore_barrier()
        return 0

    lax.fori_loop(0, KSTEPS, kstep, 0)


_prop = pl.kernel(
    _prop_body,
    out_type=jax.ShapeDtypeStruct((2 * N, D), jnp.float32),
    mesh=plsc.VectorSubcoreMesh(
        core_axis_name="c", subcore_axis_name="s",
        num_cores=1, num_subcores=16),
    scratch_types=[
        pltpu.VMEM_SHARED((N, D), jnp.float32),
        pltpu.VMEM((ECHUNK, D), jnp.float32),
        pltpu.VMEM((ECHUNK, D), jnp.float32),
        pltpu.VMEM((PACK,), jnp.int32),
        pltpu.VMEM((PACK,), jnp.int32),
        pltpu.VMEM((ECHUNK,), jnp.int32),
        pltpu.VMEM((ECHUNK,), jnp.int32),
        pltpu.VMEM((ECHUNK,), jnp.int32),
        pltpu.VMEM((ECHUNK,), jnp.int32),
        pltpu.VMEM((RCH, D), jnp.float32),
        pltpu.SemaphoreType.DMA,
        pltpu.SemaphoreType.DMA,
        pltpu.SemaphoreType.DMA,
        pltpu.SemaphoreType.DMA,
        pltpu.SemaphoreType.DMA,
        pltpu.SemaphoreType.DMA,
    ],
)


@jax.jit
def kernel(x, edge_index, edge_vals, W1, b1, W2, b2):
    h = _mlp(x, W1, b1, W2, b2)
    row = edge_index[0]
    col = edge_index[1]
    pad = E_PAD - E
    rowp = jnp.concatenate([row, jnp.zeros((pad,), jnp.int32)])
    colp = jnp.concatenate([col, jnp.zeros((pad,), jnp.int32)])
    valp = jnp.concatenate([edge_vals, jnp.zeros((pad,), jnp.float32)])
    packed = jnp.stack(
        [colp.reshape(NCHT, ECHUNK),
         rowp.reshape(NCHT, ECHUNK),
         lax.bitcast_convert_type(valp, jnp.int32).reshape(NCHT, ECHUNK)],
        axis=1).reshape(-1)
    zz = _prop(h, packed)
    return zz[:N]
